# baseline (speedup 1.0000x reference)
import functools

import numpy as np
import jax
import jax.numpy as jnp
from jax.experimental import pallas as pl

_EPS = 1e-5


def _affine(s, sq, g, b, nprev):
    mu = s / nprev
    var = sq / nprev - mu * mu
    inv = jax.lax.rsqrt(var + _EPS)
    scale = g * inv
    shift = b - mu * scale
    return scale, shift


def _stats_update(ostats_ref, acc, n):
    d = acc.shape[-1]
    ssum = jnp.sum(acc, axis=0).reshape(1, d)
    ssq = jnp.sum(acc * acc, axis=0).reshape(1, d)
    st = jnp.concatenate([ssum, ssq], axis=0)

    @pl.when(n == 0)
    def _():
        ostats_ref[...] = st

    @pl.when(n != 0)
    def _():
        ostats_ref[...] += st


def _conv1_body(x0_ref, x1_ref, w0_ref, w1_ref, out_ref, ostats_ref):
    n = pl.program_id(0)

    def prep(xref):
        xb = xref[...]
        ch, oi, _, oj, ll = xb.shape
        return xb.reshape(ch * oi * oj, ll)

    a0 = prep(x0_ref)
    a1 = prep(x1_ref)
    h = (jnp.dot(a0, w0_ref[0], preferred_element_type=jnp.float32)
         + jnp.dot(a1, w1_ref[0], preferred_element_type=jnp.float32))
    ch, oi, _, oj, _ = x0_ref.shape
    d = h.shape[-1]
    out_ref[...] = h.astype(jnp.bfloat16).reshape(ch, oi, oj, d)
    _stats_update(ostats_ref, h, n)


def _conv1_stage(xv, wmat, chunk):
    nrows, oi, _, oj, ll = xv.shape
    d = ll // 2
    nb = nrows // chunk
    blk = (chunk, oi, 1, oj, ll)
    return pl.pallas_call(
        _conv1_body,
        grid=(nb,),
        in_specs=[
            pl.BlockSpec(blk, lambda n_: (n_, 0, 0, 0, 0)),
            pl.BlockSpec(blk, lambda n_: (n_, 0, 1, 0, 0)),
            pl.BlockSpec((1, ll, d), lambda n_: (0, 0, 0)),
            pl.BlockSpec((1, ll, d), lambda n_: (1, 0, 0)),
        ],
        out_specs=[
            pl.BlockSpec((chunk, oi, oj, d), lambda n_: (n_, 0, 0, 0)),
            pl.BlockSpec((2, d), lambda n_: (0, 0)),
        ],
        out_shape=[
            jax.ShapeDtypeStruct((nrows, oi, oj, d), jnp.bfloat16),
            jax.ShapeDtypeStruct((2, d), jnp.float32),
        ],
    )(xv, xv, wmat, wmat)


def _conv_body(x_ref, w_ref, stats_ref, g_ref, b_ref, out_ref, ostats_ref,
               *, nprev, normalize):
    n = pl.program_id(0)
    xb = x_ref[...]
    ch, s, _, d = xb.shape
    if normalize:
        scale, shift = _affine(stats_ref[0:1, :], stats_ref[1:2, :],
                               g_ref[...], b_ref[...], nprev)
        xb = jnp.maximum(xb * scale.astype(jnp.bfloat16)
                         + shift.astype(jnp.bfloat16), 0.0)
    xb = xb.astype(jnp.bfloat16)
    o = s // 2
    xr = xb.reshape(ch, o, 2, o, 2, d)
    acc = None
    for di in range(2):
        for dj in range(2):
            sm = xr[:, :, di, :, dj, :].reshape(ch * o * o, d)
            p = jnp.dot(sm, w_ref[2 * di + dj],
                        preferred_element_type=jnp.float32)
            acc = p if acc is None else acc + p
    out_ref[...] = acc.astype(jnp.bfloat16).reshape(ch, o, o, d)
    ssum = jnp.sum(acc, axis=0).reshape(1, d)
    ssq = jnp.sum(acc * acc, axis=0).reshape(1, d)
    st = jnp.concatenate([ssum, ssq], axis=0)

    @pl.when(n == 0)
    def _():
        ostats_ref[...] = st

    @pl.when(n != 0)
    def _():
        ostats_ref[...] += st


def _conv_stage(h, wmat, stats, g, b, chunk, nprev, normalize):
    nrows, s, _, d = h.shape
    o = s // 2
    nb = nrows // chunk
    return pl.pallas_call(
        functools.partial(_conv_body, nprev=float(nprev), normalize=normalize),
        grid=(nb,),
        in_specs=[
            pl.BlockSpec((chunk, s, s, d), lambda n_: (n_, 0, 0, 0)),
            pl.BlockSpec((4, d, d), lambda n_: (0, 0, 0)),
            pl.BlockSpec((2, d), lambda n_: (0, 0)),
            pl.BlockSpec((1, d), lambda n_: (0, 0)),
            pl.BlockSpec((1, d), lambda n_: (0, 0)),
        ],
        out_specs=[
            pl.BlockSpec((chunk, o, o, d), lambda n_: (n_, 0, 0, 0)),
            pl.BlockSpec((2, d), lambda n_: (0, 0)),
        ],
        out_shape=[
            jax.ShapeDtypeStruct((nrows, o, o, d), jnp.bfloat16),
            jax.ShapeDtypeStruct((2, d), jnp.float32),
        ],
    )(h, wmat, stats, g, b)


def _head_body(h_ref, stats_ref, g_ref, b_ref, fwt_ref, fb_ref, cos_ref,
               sin_ref, w2_ref, sel_ref, gates_ref, load_ref, *, nprev, dmod):
    scale, shift = _affine(stats_ref[0:1, :], stats_ref[1:2, :],
                           g_ref[...], b_ref[...], nprev)
    h = h_ref[...].astype(jnp.float32) * scale + shift
    h = jnp.maximum(h, 0.0)
    g = jnp.dot(h, fwt_ref[...], preferred_element_type=jnp.float32)
    g = g + fb_ref[...]
    re = jnp.dot(cos_ref[...], g, preferred_element_type=jnp.float32)
    im = jnp.dot(sin_ref[...], g, preferred_element_type=jnp.float32)
    amp = jnp.sqrt(re * re + im * im)
    rs = jnp.sum(amp, axis=1, keepdims=True) * (1.0 / dmod)
    t = rs * w2_ref[...]
    w8 = jnp.dot(sel_ref[...], t, preferred_element_type=jnp.float32)
    iota = jax.lax.broadcasted_iota(jnp.int32, w8.shape, 1)
    big = jnp.int32(1 << 30)
    m1 = jnp.max(w8, axis=1, keepdims=True)
    i1 = jnp.min(jnp.where(w8 == m1, iota, big), axis=1, keepdims=True)
    wm = jnp.where(iota == i1, -jnp.inf, w8)
    m2 = jnp.max(wm, axis=1, keepdims=True)
    i2 = jnp.min(jnp.where(wm == m2, iota, big), axis=1, keepdims=True)
    e = jnp.exp(m2 - m1)
    z = 1.0 + e
    gates = jnp.where(iota == i1, 1.0 / z, jnp.where(iota == i2, e / z, 0.0))
    gates_ref[...] = gates
    load_ref[...] = jnp.sum((gates > 0.0).astype(jnp.int32), axis=0,
                            keepdims=True)


def kernel(x, training, conv_w, conv_b, bn_g, bn_b, fuse_w, fuse_b, w_gate,
           w_noise):
    b, t, c, hh, ww, d = x.shape
    n = b * t * c
    nf, nseg = w_gate.shape

    wmats = [conv_w[i].transpose(2, 3, 1, 0).reshape(4, d, d)
             .astype(jnp.bfloat16) for i in range(conv_w.shape[0])]
    w1m = conv_w[0].transpose(2, 3, 1, 0).reshape(2, 2 * d, d)\
        .astype(jnp.bfloat16)

    xv = (x.astype(jnp.bfloat16).reshape(n, hh, ww // 2, 2 * d)
          .reshape(n, hh // 2, 2, ww // 2, 2 * d))
    h, stats = _conv1_stage(xv, w1m, chunk=64)
    spatial = hh // 2
    chunks = {8: 64, 4: 256, 2: 512}
    for i in (1, 2, 3):
        h, stats = _conv_stage(h, wmats[i], stats, bn_g[i - 1][None],
                               bn_b[i - 1][None], chunk=min(chunks[spatial], n),
                               nprev=float(n * spatial * spatial),
                               normalize=True)
        spatial //= 2

    kk = np.arange(1, nf + 1, dtype=np.float64)
    tt = np.arange(t, dtype=np.float64)
    ang = 2.0 * np.pi * np.outer(kk, tt) / t
    cosm = (np.cos(ang) / np.sqrt(t)).astype(np.float32)
    sinm = (np.sin(ang) / np.sqrt(t)).astype(np.float32)
    cosb = np.zeros((b * nf, n), np.float32)
    sinb = np.zeros((b * nf, n), np.float32)
    selm = np.zeros((b, b * nf), np.float32)
    for bi in range(b):
        cosb[bi * nf:(bi + 1) * nf, bi * t:(bi + 1) * t] = cosm
        sinb[bi * nf:(bi + 1) * nf, bi * t:(bi + 1) * t] = sinm
        selm[bi, bi * nf:(bi + 1) * nf] = 1.0
    w2 = jnp.tile(w_gate, (b, 1))

    gates, load = pl.pallas_call(
        functools.partial(_head_body, nprev=float(n), dmod=float(d)),
        out_shape=[
            jax.ShapeDtypeStruct((b, nseg), jnp.float32),
            jax.ShapeDtypeStruct((1, nseg), jnp.int32),
        ],
    )(h.reshape(n, d), stats, bn_g[-1][None], bn_b[-1][None], fuse_w.T,
      fuse_b[None], jnp.asarray(cosb), jnp.asarray(sinb), w2,
      jnp.asarray(selm))
    return gates, load.reshape(nseg)


# speedup vs baseline: 2.9072x; 2.9072x over previous
import functools

import numpy as np
import jax
import jax.numpy as jnp
from jax.experimental import pallas as pl

_EPS = 1e-5


def _affine(s, sq, g, b, nprev):
    mu = s / nprev
    var = sq / nprev - mu * mu
    inv = jax.lax.rsqrt(var + _EPS)
    scale = g * inv
    shift = b - mu * scale
    return scale, shift


def _conv_body(x0_ref, x1_ref, w_ref, stats_ref, g_ref, b_ref, out_ref,
               ostats_ref, *, nprev, normalize):
    n = pl.program_id(0)
    if normalize:
        scale, shift = _affine(stats_ref[0:1, :], stats_ref[1:2, :],
                               g_ref[...], b_ref[...], nprev)
        sc = scale.astype(jnp.bfloat16)
        sh = shift.astype(jnp.bfloat16)

    def prep(xref):
        xb = xref[...]
        ch, o, _, s, d = xb.shape
        xm = xb.reshape(ch * o, s, d)
        if normalize:
            xm = jnp.maximum(xm * sc + sh, 0.0)
        return xm.astype(jnp.bfloat16).reshape(ch * o * (s // 2), 2 * d)

    halves = (prep(x0_ref), prep(x1_ref))
    d = halves[0].shape[-1] // 2
    acc = None
    for di in range(2):
        p = jnp.dot(halves[di], w_ref[di],
                    preferred_element_type=jnp.float32)
        acc = p if acc is None else acc + p
    ch, o = x0_ref.shape[0], x0_ref.shape[1]
    out_ref[...] = acc.astype(jnp.bfloat16).reshape(ch, o, o, d)
    ssum = jnp.sum(acc, axis=0).reshape(1, d)
    ssq = jnp.sum(acc * acc, axis=0).reshape(1, d)
    st = jnp.concatenate([ssum, ssq], axis=0)

    @pl.when(n == 0)
    def _():
        ostats_ref[...] = st

    @pl.when(n != 0)
    def _():
        ostats_ref[...] += st


def _conv_stage(h, wmat, stats, g, b, chunk, nprev, normalize):
    nrows, s, _, d = h.shape
    o = s // 2
    nb = nrows // chunk
    hv = h.reshape(nrows, o, 2, s, d)
    blk = (chunk, o, 1, s, d)
    return pl.pallas_call(
        functools.partial(_conv_body, nprev=float(nprev), normalize=normalize),
        grid=(nb,),
        in_specs=[
            pl.BlockSpec(blk, lambda n_: (n_, 0, 0, 0, 0)),
            pl.BlockSpec(blk, lambda n_: (n_, 0, 1, 0, 0)),
            pl.BlockSpec((2, 2 * d, d), lambda n_: (0, 0, 0)),
            pl.BlockSpec((2, d), lambda n_: (0, 0)),
            pl.BlockSpec((1, d), lambda n_: (0, 0)),
            pl.BlockSpec((1, d), lambda n_: (0, 0)),
        ],
        out_specs=[
            pl.BlockSpec((chunk, o, o, d), lambda n_: (n_, 0, 0, 0)),
            pl.BlockSpec((2, d), lambda n_: (0, 0)),
        ],
        out_shape=[
            jax.ShapeDtypeStruct((nrows, o, o, d), jnp.bfloat16),
            jax.ShapeDtypeStruct((2, d), jnp.float32),
        ],
    )(hv, hv, wmat, stats, g, b)


def _head_body(h_ref, stats_ref, g_ref, b_ref, fwt_ref, fb_ref, cos_ref,
               sin_ref, w2_ref, sel_ref, gates_ref, load_ref, *, nprev, dmod):
    scale, shift = _affine(stats_ref[0:1, :], stats_ref[1:2, :],
                           g_ref[...], b_ref[...], nprev)
    h = h_ref[...].astype(jnp.float32) * scale + shift
    h = jnp.maximum(h, 0.0)
    g = jnp.dot(h, fwt_ref[...], preferred_element_type=jnp.float32)
    g = g + fb_ref[...]
    re = jnp.dot(cos_ref[...], g, preferred_element_type=jnp.float32)
    im = jnp.dot(sin_ref[...], g, preferred_element_type=jnp.float32)
    amp = jnp.sqrt(re * re + im * im)
    rs = jnp.sum(amp, axis=1, keepdims=True) * (1.0 / dmod)
    t = rs * w2_ref[...]
    w8 = jnp.dot(sel_ref[...], t, preferred_element_type=jnp.float32)
    iota = jax.lax.broadcasted_iota(jnp.int32, w8.shape, 1)
    big = jnp.int32(1 << 30)
    m1 = jnp.max(w8, axis=1, keepdims=True)
    i1 = jnp.min(jnp.where(w8 == m1, iota, big), axis=1, keepdims=True)
    wm = jnp.where(iota == i1, -jnp.inf, w8)
    m2 = jnp.max(wm, axis=1, keepdims=True)
    i2 = jnp.min(jnp.where(wm == m2, iota, big), axis=1, keepdims=True)
    e = jnp.exp(m2 - m1)
    z = 1.0 + e
    gates = jnp.where(iota == i1, 1.0 / z, jnp.where(iota == i2, e / z, 0.0))
    gates_ref[...] = gates
    load_ref[...] = jnp.sum((gates > 0.0).astype(jnp.int32), axis=0,
                            keepdims=True)


def kernel(x, training, conv_w, conv_b, bn_g, bn_b, fuse_w, fuse_b, w_gate,
           w_noise):
    b, t, c, hh, ww, d = x.shape
    n = b * t * c
    nf, nseg = w_gate.shape

    wmats = [conv_w[i].transpose(2, 3, 1, 0).reshape(2, 2 * d, d)
             .astype(jnp.bfloat16) for i in range(conv_w.shape[0])]

    zstats = jnp.zeros((2, d), jnp.float32)
    ones = jnp.ones((1, d), jnp.float32)
    zeros = jnp.zeros((1, d), jnp.float32)

    h = x.reshape(n, hh, ww, d)
    h, stats = _conv_stage(h, wmats[0], zstats, ones, zeros,
                           chunk=64, nprev=1.0, normalize=False)
    spatial = hh // 2
    chunks = {8: 128, 4: 512, 2: 512}
    for i in (1, 2, 3):
        h, stats = _conv_stage(h, wmats[i], stats, bn_g[i - 1][None],
                               bn_b[i - 1][None], chunk=min(chunks[spatial], n),
                               nprev=float(n * spatial * spatial),
                               normalize=True)
        spatial //= 2

    kk = np.arange(1, nf + 1, dtype=np.float64)
    tt = np.arange(t, dtype=np.float64)
    ang = 2.0 * np.pi * np.outer(kk, tt) / t
    cosm = (np.cos(ang) / np.sqrt(t)).astype(np.float32)
    sinm = (np.sin(ang) / np.sqrt(t)).astype(np.float32)
    cosb = np.zeros((b * nf, n), np.float32)
    sinb = np.zeros((b * nf, n), np.float32)
    selm = np.zeros((b, b * nf), np.float32)
    for bi in range(b):
        cosb[bi * nf:(bi + 1) * nf, bi * t:(bi + 1) * t] = cosm
        sinb[bi * nf:(bi + 1) * nf, bi * t:(bi + 1) * t] = sinm
        selm[bi, bi * nf:(bi + 1) * nf] = 1.0
    w2 = jnp.tile(w_gate, (b, 1))

    gates, load = pl.pallas_call(
        functools.partial(_head_body, nprev=float(n), dmod=float(d)),
        out_shape=[
            jax.ShapeDtypeStruct((b, nseg), jnp.float32),
            jax.ShapeDtypeStruct((1, nseg), jnp.int32),
        ],
    )(h.reshape(n, d), stats, bn_g[-1][None], bn_b[-1][None], fuse_w.T,
      fuse_b[None], jnp.asarray(cosb), jnp.asarray(sinb), w2,
      jnp.asarray(selm))
    return gates, load.reshape(nseg)
